# baseline (speedup 1.0000x reference)
import functools

import jax
import jax.numpy as jnp
from jax import lax
from jax.experimental import pallas as pl
from jax.experimental.pallas import tpu as pltpu
from jax.experimental.pallas import tpu_sc as plsc

L = 16
N_IN = 5000
N_PAD = 5120
NB = N_PAD // L
B_IMG = 4
NSLICE = 8
VB = NB // NSLICE
CONF_THRESH = 0.01
IOU_THRESH = 0.5
MAX_WH = 4096.0
MARGIN = 1.0625
NEG_BIG = -3.0e38
OUTER_CAP = 320
INNER_SLOTS = 16


def _nms_body(ff_h, pm_h, tb_h, out_h,
              ff_v, pm_v,
              X1O, Y1O, X2O, Y2O, AREA, CONF, IDXT, VAL,
              keep_a, keep_b, chg_v, chg_new, tb_v, out_v, dbuf, flag_v,
              sh_keep, sh_chg, sh_flag,
              lo_s, hi_s, gmin_s, gmax_s, flag_s):
    cid = lax.axis_index("c")
    sid = lax.axis_index("s")
    wid = sid * 2 + cid
    img = wid % B_IMG
    r = wid // B_IMG
    slot = img // 2

    pltpu.sync_copy(ff_h.at[pl.ds(img * 6 * N_PAD, 6 * N_PAD)], ff_v)
    pltpu.sync_copy(pm_h.at[pl.ds(img * N_PAD, N_PAD)], pm_v)
    pltpu.sync_copy(tb_h, tb_v)

    tbv = tb_v[...]
    tx1 = tbv[0]
    ty1 = tbv[1]
    tx2 = tbv[2]
    ty2 = tbv[3]
    t_area = (tx2 - tx1) * (ty2 - ty1)

    def derive_blk(v, _):
        o = v * L
        idx = pm_v[pl.ds(o, L)]
        gx1 = ff_v[pl.ds(0 * N_PAD + o, L)]
        gy1 = ff_v[pl.ds(1 * N_PAD + o, L)]
        gx2 = ff_v[pl.ds(2 * N_PAD + o, L)]
        gy2 = ff_v[pl.ds(3 * N_PAD + o, L)]
        gcf = ff_v[pl.ds(4 * N_PAD + o, L)]
        gcl = ff_v[pl.ds(5 * N_PAD + o, L)]
        off = gcl * MAX_WH
        x1o = gx1 + off
        y1o = gy1 + off
        x2o = gx2 + off
        y2o = gy2 + off
        X1O[pl.ds(o, L)] = x1o
        Y1O[pl.ds(o, L)] = y1o
        X2O[pl.ds(o, L)] = x2o
        Y2O[pl.ds(o, L)] = y2o
        AREA[pl.ds(o, L)] = (x2o - x1o) * (y2o - y1o)
        CONF[pl.ds(o, L)] = gcf
        IDXT[pl.ds(o, L)] = idx
        gmin_s[v] = x1o[0]
        gmax_s[v] = x1o[L - 1]
        ltx = jnp.maximum(tx1, gx1)
        lty = jnp.maximum(ty1, gy1)
        rbx = jnp.minimum(tx2, gx2)
        rby = jnp.minimum(ty2, gy2)
        inter = jnp.maximum(rbx - ltx, 0.0) * jnp.maximum(rby - lty, 0.0)
        b_area = (gx2 - gx1) * (gy2 - gy1)
        iou_t = inter / (t_area + b_area - inter + 1e-9)
        VAL[pl.ds(o, L)] = iou_t * gcf
        mblk = jnp.where(gcf > CONF_THRESH, 1.0, 0.0)
        keep_a[pl.ds(o, L)] = mblk
        keep_b[pl.ds(o, L)] = mblk
        chg_v[pl.ds(o, L)] = jnp.full((L,), 1.0, jnp.float32)
        return 0

    lax.fori_loop(0, NB, derive_blk, 0)

    def win_blk(t, _):
        v = r * VB + t
        cut_lo = gmin_s[v] - MARGIN
        cut_hi = gmax_s[v] + MARGIN
        p_lo = jnp.int32(0)
        p_hi = jnp.int32(0)
        for sh in (256, 128, 64, 32, 16, 8, 4, 2, 1):
            cand = p_lo + sh
            ok = ((cand <= NB)
                  & (gmax_s[jnp.minimum(cand - 1, NB - 1)] <= cut_lo))
            p_lo = jnp.where(ok, cand, p_lo)
            cand2 = p_hi + sh
            ok2 = ((cand2 <= NB)
                   & (gmin_s[jnp.minimum(cand2 - 1, NB - 1)] < cut_hi))
            p_hi = jnp.where(ok2, cand2, p_hi)
        lo_s[v] = p_lo
        hi_s[v] = jnp.maximum(p_hi, p_lo)
        return 0

    lax.fori_loop(0, VB, win_blk, 0)

    def lane_sum(vec):
        s = vec[0]
        for k in range(1, L):
            s = s + vec[k]
        return s

    def lane_max(vec):
        s = vec[0]
        for k in range(1, L):
            s = jnp.maximum(s, vec[k])
        return s

    def blk(t, dvec):
        v = r * VB + t
        o = v * L

        def wchk(g, acc):
            return jnp.maximum(acc, chg_v[pl.ds(g * L, L)])

        wch16 = lax.fori_loop(lo_s[v], hi_s[v], wchk,
                              jnp.zeros((L,), jnp.float32))
        wch = lane_max(wch16)
        jx1 = X1O[pl.ds(o, L)]
        jy1 = Y1O[pl.ds(o, L)]
        jx2 = X2O[pl.ds(o, L)]
        jy2 = Y2O[pl.ds(o, L)]
        jar = AREA[pl.ds(o, L)]
        jcf = CONF[pl.ds(o, L)]
        jid = IDXT[pl.ds(o, L)]

        def inner(g, supp):
            go = g * L
            ix1 = X1O[pl.ds(go, L)]
            iy1 = Y1O[pl.ds(go, L)]
            ix2 = X2O[pl.ds(go, L)]
            iy2 = Y2O[pl.ds(go, L)]
            iar = AREA[pl.ds(go, L)]
            icf = CONF[pl.ds(go, L)]
            iid = IDXT[pl.ds(go, L)]
            ikp = keep_a[pl.ds(go, L)]
            for k in range(L):
                ltx = jnp.maximum(ix1[k], jx1)
                lty = jnp.maximum(iy1[k], jy1)
                rbx = jnp.minimum(ix2[k], jx2)
                rby = jnp.minimum(iy2[k], jy2)
                inter = (jnp.maximum(rbx - ltx, 0.0)
                         * jnp.maximum(rby - lty, 0.0))
                iou = inter / (iar[k] + jar - inter + 1e-9)
                s = iou > IOU_THRESH
                higher = ((icf[k] > jcf)
                          | ((icf[k] == jcf) & (iid[k] < jid)))
                supp = jnp.maximum(supp, jnp.where(s & higher, ikp[k], 0.0))
            return supp

        n_hi = jnp.where(wch > 0.5, hi_s[v], lo_s[v])
        supp = lax.fori_loop(lo_s[v], n_hi, inner,
                             jnp.zeros((L,), jnp.float32))
        mblk = jnp.where(jcf > CONF_THRESH, 1.0, 0.0)
        computed = jnp.where(supp > 0.5, 0.0, mblk)
        prev = keep_b[pl.ds(o, L)]
        newv = jnp.where(wch > 0.5, computed, prev)
        keep_b[pl.ds(o, L)] = newv
        oldv = keep_a[pl.ds(o, L)]
        dchg = jnp.abs(newv - oldv)
        chg_new[pl.ds(o, L)] = dchg
        return dvec + dchg

    flag_s[0] = jnp.float32(1.0)

    def pass_slot(q, _):
        ch = flag_s[0]
        n_eff = jnp.where(ch > 0.5, jnp.int32(VB), jnp.int32(0))
        d16 = lax.fori_loop(0, n_eff, blk, jnp.zeros((L,), jnp.float32))

        @pl.when(ch > 0.5)
        def _():
            dbuf[...] = jnp.full((L,), 1.0, jnp.float32) * lane_sum(d16)
            own = r * VB * L
            pltpu.sync_copy(keep_b.at[pl.ds(own, VB * L)],
                            sh_keep.at[slot, pl.ds(own, VB * L)])
            pltpu.sync_copy(chg_new.at[pl.ds(own, VB * L)],
                            sh_chg.at[slot, pl.ds(own, VB * L)])
            pltpu.sync_copy(dbuf.at[pl.ds(0, 8)],
                            sh_flag.at[pl.ds(sid * 8, 8)])
            plsc.subcore_barrier()
            pltpu.sync_copy(sh_keep.at[slot], keep_a)
            pltpu.sync_copy(sh_chg.at[slot], chg_v)
            pltpu.sync_copy(sh_flag, flag_v)
            acc = flag_v[pl.ds(0, L)]
            for w in range(1, 8):
                acc = acc + flag_v[pl.ds(w * L, L)]
            flag_s[0] = lane_sum(acc)
            plsc.subcore_barrier()

        return 0

    def outer_round(w, _):
        ch = flag_s[0]
        n_in = jnp.where(ch > 0.5, jnp.int32(INNER_SLOTS), jnp.int32(0))
        lax.fori_loop(0, n_in, pass_slot, 0)
        return 0

    lax.fori_loop(0, OUTER_CAP, outer_round, 0)

    @pl.when(wid < B_IMG)
    def _():
        def red_blk(v, carry):
            best16, any16 = carry
            o = v * L
            kv = keep_a[pl.ds(o, L)]
            vv = VAL[pl.ds(o, L)]
            best16 = jnp.maximum(best16, jnp.where(kv > 0.5, vv, NEG_BIG))
            any16 = jnp.maximum(any16, kv)
            return (best16, any16)

        best16, any16 = lax.fori_loop(
            0, NB, red_blk,
            (jnp.full((L,), NEG_BIG, jnp.float32),
             jnp.zeros((L,), jnp.float32)))
        score = jnp.where(lane_max(any16) > 0.5, lane_max(best16), 0.0)
        out_v[...] = jnp.full((L,), 1.0, jnp.float32) * score
        pltpu.sync_copy(out_v, out_h.at[pl.ds(wid * L, L)])


@jax.jit
def _od2_score(ff, perm, tbp):
    mesh = plsc.VectorSubcoreMesh(core_axis_name="c", subcore_axis_name="s")
    fn = functools.partial(
        pl.kernel,
        mesh=mesh,
        out_type=jax.ShapeDtypeStruct((B_IMG * L,), jnp.float32),
        scratch_types=[
            pltpu.VMEM((6 * N_PAD,), jnp.float32),
            pltpu.VMEM((N_PAD,), jnp.int32),
            pltpu.VMEM((N_PAD,), jnp.float32),
            pltpu.VMEM((N_PAD,), jnp.float32),
            pltpu.VMEM((N_PAD,), jnp.float32),
            pltpu.VMEM((N_PAD,), jnp.float32),
            pltpu.VMEM((N_PAD,), jnp.float32),
            pltpu.VMEM((N_PAD,), jnp.float32),
            pltpu.VMEM((N_PAD,), jnp.int32),
            pltpu.VMEM((N_PAD,), jnp.float32),
            pltpu.VMEM((N_PAD,), jnp.float32),
            pltpu.VMEM((N_PAD,), jnp.float32),
            pltpu.VMEM((N_PAD,), jnp.float32),
            pltpu.VMEM((N_PAD,), jnp.float32),
            pltpu.VMEM((L,), jnp.float32),
            pltpu.VMEM((L,), jnp.float32),
            pltpu.VMEM((L,), jnp.float32),
            pltpu.VMEM((NSLICE * L,), jnp.float32),
            pltpu.VMEM_SHARED((2, N_PAD), jnp.float32),
            pltpu.VMEM_SHARED((2, N_PAD), jnp.float32),
            pltpu.VMEM_SHARED((NSLICE * L,), jnp.float32),
            pltpu.SMEM((NB,), jnp.int32),
            pltpu.SMEM((NB,), jnp.int32),
            pltpu.SMEM((NB,), jnp.float32),
            pltpu.SMEM((NB,), jnp.float32),
            pltpu.SMEM((8,), jnp.float32),
        ],
    )(_nms_body)
    return fn(ff, perm, tbp)


def kernel(prediction, target_bbox):
    assert prediction.shape == (B_IMG, N_IN, 6)
    pad = N_PAD - N_IN
    x1 = jnp.pad(prediction[..., 0], ((0, 0), (0, pad)))
    y1 = jnp.pad(prediction[..., 1], ((0, 0), (0, pad)))
    x2 = jnp.pad(prediction[..., 2], ((0, 0), (0, pad)))
    y2 = jnp.pad(prediction[..., 3], ((0, 0), (0, pad)))
    cf = jnp.pad(prediction[..., 4], ((0, 0), (0, pad)), constant_values=-1.0)
    cl = jnp.pad(prediction[..., 5], ((0, 0), (0, pad)), constant_values=1.0e6)
    key = x1 + cl * MAX_WH
    iota = jnp.broadcast_to(
        jnp.arange(N_PAD, dtype=jnp.int32), (B_IMG, N_PAD))
    _, perm = lax.sort((key, iota), num_keys=1, is_stable=False)
    fields = jnp.stack([x1, y1, x2, y2, cf, cl], axis=1)
    fields = jnp.take_along_axis(fields, perm[:, None, :], axis=2)
    tbp = jnp.pad(target_bbox[0].astype(jnp.float32), (0, L - 4))
    out = _od2_score(fields.reshape(-1), perm.reshape(-1), tbp)
    return out.reshape(B_IMG, L)[:, 0]


# speedup vs baseline: 7.8008x; 7.8008x over previous
import functools

import jax
import jax.numpy as jnp
from jax import lax
from jax.experimental import pallas as pl
from jax.experimental.pallas import tpu as pltpu
from jax.experimental.pallas import tpu_sc as plsc

L = 16
N_IN = 5000
N_PAD = 5120
NB = N_PAD // L
B_IMG = 4
NSLICE = 8
VB = NB // NSLICE
CONF_THRESH = 0.01
IOU_THRESH = 0.5
MAX_WH = 4096.0
MARGIN = 1.0625
NEG_BIG = -3.0e38
OUTER_CAP = 320
INNER_SLOTS = 16


def _nms_body(ff_h, pm_h, tb_h, out_h,
              ff_v, pm_v,
              X1O, Y1O, X2O, Y2O, AREA, CONF, IDXT, VAL,
              keep_a, keep_b, chg_v, chg_new, tb_v, out_v, dbuf, flag_v,
              sh_keep, sh_chg, sh_flag,
              lo_s, hi_s, gmin_s, gmax_s, flag_s):
    cid = lax.axis_index("c")
    sid = lax.axis_index("s")
    wid = sid * 2 + cid
    img = wid % B_IMG
    r = wid // B_IMG
    slot = img // 2

    pltpu.sync_copy(ff_h.at[pl.ds(img * 6 * N_PAD, 6 * N_PAD)], ff_v)
    pltpu.sync_copy(pm_h.at[pl.ds(img * N_PAD, N_PAD)], pm_v)
    pltpu.sync_copy(tb_h, tb_v)

    tbv = tb_v[...]
    tx1 = tbv[0]
    ty1 = tbv[1]
    tx2 = tbv[2]
    ty2 = tbv[3]
    t_area = (tx2 - tx1) * (ty2 - ty1)

    def derive_blk(v, _):
        o = v * L
        idx = pm_v[pl.ds(o, L)]
        gx1 = ff_v[pl.ds(0 * N_PAD + o, L)]
        gy1 = ff_v[pl.ds(1 * N_PAD + o, L)]
        gx2 = ff_v[pl.ds(2 * N_PAD + o, L)]
        gy2 = ff_v[pl.ds(3 * N_PAD + o, L)]
        gcf = ff_v[pl.ds(4 * N_PAD + o, L)]
        gcl = ff_v[pl.ds(5 * N_PAD + o, L)]
        off = gcl * MAX_WH
        x1o = gx1 + off
        y1o = gy1 + off
        x2o = gx2 + off
        y2o = gy2 + off
        X1O[pl.ds(o, L)] = x1o
        Y1O[pl.ds(o, L)] = y1o
        X2O[pl.ds(o, L)] = x2o
        Y2O[pl.ds(o, L)] = y2o
        AREA[pl.ds(o, L)] = (x2o - x1o) * (y2o - y1o)
        CONF[pl.ds(o, L)] = gcf
        IDXT[pl.ds(o, L)] = idx
        gmin_s[v] = x1o[0]
        gmax_s[v] = x1o[L - 1]
        ltx = jnp.maximum(tx1, gx1)
        lty = jnp.maximum(ty1, gy1)
        rbx = jnp.minimum(tx2, gx2)
        rby = jnp.minimum(ty2, gy2)
        inter = jnp.maximum(rbx - ltx, 0.0) * jnp.maximum(rby - lty, 0.0)
        b_area = (gx2 - gx1) * (gy2 - gy1)
        iou_t = inter / (t_area + b_area - inter + 1e-9)
        VAL[pl.ds(o, L)] = iou_t * gcf
        mblk = jnp.where(gcf > CONF_THRESH, 1.0, 0.0)
        keep_a[pl.ds(o, L)] = mblk
        keep_b[pl.ds(o, L)] = mblk
        chg_v[pl.ds(o, L)] = jnp.full((L,), 1.0, jnp.float32)
        return 0

    lax.fori_loop(0, NB, derive_blk, 0)

    def win_blk(t, _):
        v = r * VB + t
        cut_lo = gmin_s[v] - MARGIN
        cut_hi = gmax_s[v] + MARGIN
        p_lo = jnp.int32(0)
        p_hi = jnp.int32(0)
        for sh in (256, 128, 64, 32, 16, 8, 4, 2, 1):
            cand = p_lo + sh
            ok = ((cand <= NB)
                  & (gmax_s[jnp.minimum(cand - 1, NB - 1)] <= cut_lo))
            p_lo = jnp.where(ok, cand, p_lo)
            cand2 = p_hi + sh
            ok2 = ((cand2 <= NB)
                   & (gmin_s[jnp.minimum(cand2 - 1, NB - 1)] < cut_hi))
            p_hi = jnp.where(ok2, cand2, p_hi)
        lo_s[v] = p_lo
        hi_s[v] = jnp.maximum(p_hi, p_lo)
        return 0

    lax.fori_loop(0, VB, win_blk, 0)

    def lane_sum(vec):
        s = vec[0]
        for k in range(1, L):
            s = s + vec[k]
        return s

    def lane_max(vec):
        s = vec[0]
        for k in range(1, L):
            s = jnp.maximum(s, vec[k])
        return s

    def blk(t, dvec):
        v = r * VB + t
        o = v * L

        def wchk(g, acc):
            return jnp.maximum(acc, chg_v[pl.ds(g * L, L)])

        wch16 = lax.fori_loop(lo_s[v], hi_s[v], wchk,
                              jnp.zeros((L,), jnp.float32))
        wch = lane_max(wch16)
        jx1 = X1O[pl.ds(o, L)]
        jy1 = Y1O[pl.ds(o, L)]
        jx2 = X2O[pl.ds(o, L)]
        jy2 = Y2O[pl.ds(o, L)]
        jar = AREA[pl.ds(o, L)]
        jcf = CONF[pl.ds(o, L)]
        jid = IDXT[pl.ds(o, L)]

        def inner(g, supp):
            go = g * L
            ix1 = X1O[pl.ds(go, L)]
            iy1 = Y1O[pl.ds(go, L)]
            ix2 = X2O[pl.ds(go, L)]
            iy2 = Y2O[pl.ds(go, L)]
            iar = AREA[pl.ds(go, L)]
            icf = CONF[pl.ds(go, L)]
            iid = IDXT[pl.ds(go, L)]
            ikp = keep_a[pl.ds(go, L)]
            for k in range(L):
                ltx = jnp.maximum(ix1[k], jx1)
                lty = jnp.maximum(iy1[k], jy1)
                rbx = jnp.minimum(ix2[k], jx2)
                rby = jnp.minimum(iy2[k], jy2)
                inter = (jnp.maximum(rbx - ltx, 0.0)
                         * jnp.maximum(rby - lty, 0.0))
                iou = inter / (iar[k] + jar - inter + 1e-9)
                s = iou > IOU_THRESH
                higher = ((icf[k] > jcf)
                          | ((icf[k] == jcf) & (iid[k] < jid)))
                supp = jnp.maximum(supp, jnp.where(s & higher, ikp[k], 0.0))
            return supp

        n_hi = jnp.where(wch > 0.5, hi_s[v], lo_s[v])
        supp = lax.fori_loop(lo_s[v], n_hi, inner,
                             jnp.zeros((L,), jnp.float32))
        mblk = jnp.where(jcf > CONF_THRESH, 1.0, 0.0)
        computed = jnp.where(supp > 0.5, 0.0, mblk)
        prev = keep_b[pl.ds(o, L)]
        newv = jnp.where(wch > 0.5, computed, prev)
        keep_b[pl.ds(o, L)] = newv
        oldv = keep_a[pl.ds(o, L)]
        dchg = jnp.abs(newv - oldv)
        chg_new[pl.ds(o, L)] = dchg
        return dvec + dchg

    flag_s[0] = jnp.float32(1.0)

    def pass_slot(q, _):
        ch = flag_s[0]
        n_eff = jnp.where(ch > 0.5, jnp.int32(VB), jnp.int32(0))
        d16 = lax.fori_loop(0, n_eff, blk, jnp.zeros((L,), jnp.float32))

        @pl.when(ch > 0.5)
        def _():
            dbuf[...] = jnp.full((L,), 1.0, jnp.float32) * lane_sum(d16)
            own = r * VB * L
            pltpu.sync_copy(keep_b.at[pl.ds(own, VB * L)],
                            sh_keep.at[slot, pl.ds(own, VB * L)])
            pltpu.sync_copy(chg_new.at[pl.ds(own, VB * L)],
                            sh_chg.at[slot, pl.ds(own, VB * L)])
            pltpu.sync_copy(dbuf.at[pl.ds(0, 8)],
                            sh_flag.at[pl.ds(sid * 8, 8)])
            plsc.subcore_barrier()
            pltpu.sync_copy(sh_keep.at[slot], keep_a)
            pltpu.sync_copy(sh_chg.at[slot], chg_v)
            pltpu.sync_copy(sh_flag, flag_v)
            acc = flag_v[pl.ds(0, L)]
            for w in range(1, 8):
                acc = acc + flag_v[pl.ds(w * L, L)]
            flag_s[0] = lane_sum(acc)
            plsc.subcore_barrier()

        return 0

    def outer_round(w, _):
        ch = flag_s[0]
        n_in = jnp.where(ch > 0.5, jnp.int32(INNER_SLOTS), jnp.int32(0))
        lax.fori_loop(0, n_in, pass_slot, 0)
        return 0

    lax.fori_loop(0, OUTER_CAP, outer_round, 0)

    @pl.when(wid < B_IMG)
    def _():
        def red_blk(v, carry):
            best16, any16 = carry
            o = v * L
            kv = keep_a[pl.ds(o, L)]
            vv = VAL[pl.ds(o, L)]
            best16 = jnp.maximum(best16, jnp.where(kv > 0.5, vv, NEG_BIG))
            any16 = jnp.maximum(any16, kv)
            return (best16, any16)

        best16, any16 = lax.fori_loop(
            0, NB, red_blk,
            (jnp.full((L,), NEG_BIG, jnp.float32),
             jnp.zeros((L,), jnp.float32)))
        score = jnp.where(lane_max(any16) > 0.5, lane_max(best16), 0.0)
        out_v[...] = jnp.full((L,), 1.0, jnp.float32) * score
        pltpu.sync_copy(out_v, out_h.at[pl.ds(wid * L, L)])


@jax.jit
def _od2_score(ff, perm, tbp):
    mesh = plsc.VectorSubcoreMesh(core_axis_name="c", subcore_axis_name="s")
    fn = functools.partial(
        pl.kernel,
        mesh=mesh,
        out_type=jax.ShapeDtypeStruct((B_IMG * L,), jnp.float32),
        scratch_types=[
            pltpu.VMEM((6 * N_PAD,), jnp.float32),
            pltpu.VMEM((N_PAD,), jnp.int32),
            pltpu.VMEM((N_PAD,), jnp.float32),
            pltpu.VMEM((N_PAD,), jnp.float32),
            pltpu.VMEM((N_PAD,), jnp.float32),
            pltpu.VMEM((N_PAD,), jnp.float32),
            pltpu.VMEM((N_PAD,), jnp.float32),
            pltpu.VMEM((N_PAD,), jnp.float32),
            pltpu.VMEM((N_PAD,), jnp.int32),
            pltpu.VMEM((N_PAD,), jnp.float32),
            pltpu.VMEM((N_PAD,), jnp.float32),
            pltpu.VMEM((N_PAD,), jnp.float32),
            pltpu.VMEM((N_PAD,), jnp.float32),
            pltpu.VMEM((N_PAD,), jnp.float32),
            pltpu.VMEM((L,), jnp.float32),
            pltpu.VMEM((L,), jnp.float32),
            pltpu.VMEM((L,), jnp.float32),
            pltpu.VMEM((NSLICE * L,), jnp.float32),
            pltpu.VMEM_SHARED((2, N_PAD), jnp.float32),
            pltpu.VMEM_SHARED((2, N_PAD), jnp.float32),
            pltpu.VMEM_SHARED((NSLICE * L,), jnp.float32),
            pltpu.SMEM((NB,), jnp.int32),
            pltpu.SMEM((NB,), jnp.int32),
            pltpu.SMEM((NB,), jnp.float32),
            pltpu.SMEM((NB,), jnp.float32),
            pltpu.SMEM((8,), jnp.float32),
        ],
    )(_nms_body)
    return fn(ff, perm, tbp)


def kernel(prediction, target_bbox):
    assert prediction.shape == (B_IMG, N_IN, 6)
    pad = N_PAD - N_IN
    x1 = jnp.pad(prediction[..., 0], ((0, 0), (0, pad)))
    y1 = jnp.pad(prediction[..., 1], ((0, 0), (0, pad)))
    x2 = jnp.pad(prediction[..., 2], ((0, 0), (0, pad)))
    y2 = jnp.pad(prediction[..., 3], ((0, 0), (0, pad)))
    cf = jnp.pad(prediction[..., 4], ((0, 0), (0, pad)), constant_values=-1.0)
    cl = jnp.pad(prediction[..., 5], ((0, 0), (0, pad)), constant_values=1.0e6)
    key = x1 + cl * MAX_WH
    perm = jnp.argsort(key, axis=1).astype(jnp.int32)
    fields = jnp.stack([x1, y1, x2, y2, cf, cl], axis=1)
    fields = jnp.take_along_axis(fields, perm[:, None, :], axis=2)
    tbp = jnp.pad(target_bbox[0].astype(jnp.float32), (0, L - 4))
    out = _od2_score(fields.reshape(-1), perm.reshape(-1), tbp)
    return out.reshape(B_IMG, L)[:, 0]


# speedup vs baseline: 7.9284x; 1.0164x over previous
import functools

import jax
import jax.numpy as jnp
from jax import lax
from jax.experimental import pallas as pl
from jax.experimental.pallas import tpu as pltpu
from jax.experimental.pallas import tpu_sc as plsc

L = 16
N_IN = 5000
N_PAD = 5120
NB = N_PAD // L
B_IMG = 4
NSLICE = 8
VB = NB // NSLICE
CONF_THRESH = 0.01
IOU_THRESH = 0.5
MAX_WH = 4096.0
MARGIN = 1.0625
NEG_BIG = -3.0e38
OUTER_CAP = 320
INNER_SLOTS = 16


def _nms_body(ff_h, pm_h, tb_h, out_h,
              x1_v, y1_v, x2_v, y2_v, cf_v, cl_v, pm_v,
              X1O, Y1O, X2O, Y2O, AREA, CONF, IDXT, VAL,
              keep_a, keep_b, chg_v, chg_new, tb_v, out_v, dbuf, flag_v,
              sh_keep, sh_chg, sh_flag,
              lo_s, hi_s, gmin_s, gmax_s, flag_s):
    cid = lax.axis_index("c")
    sid = lax.axis_index("s")
    wid = sid * 2 + cid
    img = wid % B_IMG
    r = wid // B_IMG
    slot = img // 2

    fbase = img * 6 * N_PAD
    pltpu.sync_copy(ff_h.at[pl.ds(fbase + 0 * N_PAD, N_PAD)], x1_v)
    pltpu.sync_copy(ff_h.at[pl.ds(fbase + 1 * N_PAD, N_PAD)], y1_v)
    pltpu.sync_copy(ff_h.at[pl.ds(fbase + 2 * N_PAD, N_PAD)], x2_v)
    pltpu.sync_copy(ff_h.at[pl.ds(fbase + 3 * N_PAD, N_PAD)], y2_v)
    pltpu.sync_copy(ff_h.at[pl.ds(fbase + 4 * N_PAD, N_PAD)], cf_v)
    pltpu.sync_copy(ff_h.at[pl.ds(fbase + 5 * N_PAD, N_PAD)], cl_v)
    pltpu.sync_copy(pm_h.at[pl.ds(img * N_PAD, N_PAD)], pm_v)
    pltpu.sync_copy(tb_h, tb_v)

    tbv = tb_v[...]
    tx1 = tbv[0]
    ty1 = tbv[1]
    tx2 = tbv[2]
    ty2 = tbv[3]
    t_area = (tx2 - tx1) * (ty2 - ty1)

    def derive_blk(v, _):
        o = v * L
        idx = pm_v[pl.ds(o, L)]
        gx1 = x1_v[pl.ds(o, L)]
        gy1 = y1_v[pl.ds(o, L)]
        gx2 = x2_v[pl.ds(o, L)]
        gy2 = y2_v[pl.ds(o, L)]
        gcf = cf_v[pl.ds(o, L)]
        gcl = cl_v[pl.ds(o, L)]
        off = gcl * MAX_WH
        x1o = gx1 + off
        y1o = gy1 + off
        x2o = gx2 + off
        y2o = gy2 + off
        X1O[pl.ds(o, L)] = x1o
        Y1O[pl.ds(o, L)] = y1o
        X2O[pl.ds(o, L)] = x2o
        Y2O[pl.ds(o, L)] = y2o
        AREA[pl.ds(o, L)] = (x2o - x1o) * (y2o - y1o)
        CONF[pl.ds(o, L)] = gcf
        IDXT[pl.ds(o, L)] = idx
        gmin_s[v] = x1o[0]
        gmax_s[v] = x1o[L - 1]
        ltx = jnp.maximum(tx1, gx1)
        lty = jnp.maximum(ty1, gy1)
        rbx = jnp.minimum(tx2, gx2)
        rby = jnp.minimum(ty2, gy2)
        inter = jnp.maximum(rbx - ltx, 0.0) * jnp.maximum(rby - lty, 0.0)
        b_area = (gx2 - gx1) * (gy2 - gy1)
        iou_t = inter / (t_area + b_area - inter + 1e-9)
        VAL[pl.ds(o, L)] = iou_t * gcf
        mblk = jnp.where(gcf > CONF_THRESH, 1.0, 0.0)
        keep_a[pl.ds(o, L)] = mblk
        keep_b[pl.ds(o, L)] = mblk
        chg_v[pl.ds(o, L)] = jnp.full((L,), 1.0, jnp.float32)
        return 0

    lax.fori_loop(0, NB, derive_blk, 0)

    def win_blk(t, _):
        v = r * VB + t
        cut_lo = gmin_s[v] - MARGIN
        cut_hi = gmax_s[v] + MARGIN
        p_lo = jnp.int32(0)
        p_hi = jnp.int32(0)
        for sh in (256, 128, 64, 32, 16, 8, 4, 2, 1):
            cand = p_lo + sh
            ok = ((cand <= NB)
                  & (gmax_s[jnp.minimum(cand - 1, NB - 1)] <= cut_lo))
            p_lo = jnp.where(ok, cand, p_lo)
            cand2 = p_hi + sh
            ok2 = ((cand2 <= NB)
                   & (gmin_s[jnp.minimum(cand2 - 1, NB - 1)] < cut_hi))
            p_hi = jnp.where(ok2, cand2, p_hi)
        lo_s[v] = p_lo
        hi_s[v] = jnp.maximum(p_hi, p_lo)
        return 0

    lax.fori_loop(0, VB, win_blk, 0)

    def lane_sum(vec):
        s = vec[0]
        for k in range(1, L):
            s = s + vec[k]
        return s

    def lane_max(vec):
        s = vec[0]
        for k in range(1, L):
            s = jnp.maximum(s, vec[k])
        return s

    def blk(t, dvec):
        v = r * VB + t
        o = v * L

        def wchk(g, acc):
            return jnp.maximum(acc, chg_v[pl.ds(g * L, L)])

        wch16 = lax.fori_loop(lo_s[v], hi_s[v], wchk,
                              jnp.zeros((L,), jnp.float32))
        wch = lane_max(wch16)
        jx1 = X1O[pl.ds(o, L)]
        jy1 = Y1O[pl.ds(o, L)]
        jx2 = X2O[pl.ds(o, L)]
        jy2 = Y2O[pl.ds(o, L)]
        jar = AREA[pl.ds(o, L)]
        jcf = CONF[pl.ds(o, L)]
        jid = IDXT[pl.ds(o, L)]

        def inner(g, supp):
            go = g * L
            ix1 = X1O[pl.ds(go, L)]
            iy1 = Y1O[pl.ds(go, L)]
            ix2 = X2O[pl.ds(go, L)]
            iy2 = Y2O[pl.ds(go, L)]
            iar = AREA[pl.ds(go, L)]
            icf = CONF[pl.ds(go, L)]
            iid = IDXT[pl.ds(go, L)]
            ikp = keep_a[pl.ds(go, L)]
            for k in range(L):
                ltx = jnp.maximum(ix1[k], jx1)
                lty = jnp.maximum(iy1[k], jy1)
                rbx = jnp.minimum(ix2[k], jx2)
                rby = jnp.minimum(iy2[k], jy2)
                inter = (jnp.maximum(rbx - ltx, 0.0)
                         * jnp.maximum(rby - lty, 0.0))
                iou = inter / (iar[k] + jar - inter + 1e-9)
                s = iou > IOU_THRESH
                higher = ((icf[k] > jcf)
                          | ((icf[k] == jcf) & (iid[k] < jid)))
                supp = jnp.maximum(supp, jnp.where(s & higher, ikp[k], 0.0))
            return supp

        n_hi = jnp.where(wch > 0.5, hi_s[v], lo_s[v])
        supp = lax.fori_loop(lo_s[v], n_hi, inner,
                             jnp.zeros((L,), jnp.float32))
        mblk = jnp.where(jcf > CONF_THRESH, 1.0, 0.0)
        computed = jnp.where(supp > 0.5, 0.0, mblk)
        prev = keep_b[pl.ds(o, L)]
        newv = jnp.where(wch > 0.5, computed, prev)
        keep_b[pl.ds(o, L)] = newv
        oldv = keep_a[pl.ds(o, L)]
        dchg = jnp.abs(newv - oldv)
        chg_new[pl.ds(o, L)] = dchg
        return dvec + dchg

    flag_s[0] = jnp.float32(1.0)

    def pass_slot(q, _):
        ch = flag_s[0]
        n_eff = jnp.where(ch > 0.5, jnp.int32(VB), jnp.int32(0))
        d16 = lax.fori_loop(0, n_eff, blk, jnp.zeros((L,), jnp.float32))

        @pl.when(ch > 0.5)
        def _():
            dbuf[...] = jnp.full((L,), 1.0, jnp.float32) * lane_sum(d16)
            own = r * VB * L
            pltpu.sync_copy(keep_b.at[pl.ds(own, VB * L)],
                            sh_keep.at[slot, pl.ds(own, VB * L)])
            pltpu.sync_copy(chg_new.at[pl.ds(own, VB * L)],
                            sh_chg.at[slot, pl.ds(own, VB * L)])
            pltpu.sync_copy(dbuf.at[pl.ds(0, 8)],
                            sh_flag.at[pl.ds(sid * 8, 8)])
            plsc.subcore_barrier()
            pltpu.sync_copy(sh_keep.at[slot], keep_a)
            pltpu.sync_copy(sh_chg.at[slot], chg_v)
            pltpu.sync_copy(sh_flag, flag_v)
            acc = flag_v[pl.ds(0, L)]
            for w in range(1, 8):
                acc = acc + flag_v[pl.ds(w * L, L)]
            flag_s[0] = lane_sum(acc)
            plsc.subcore_barrier()

        return 0

    def outer_round(w, _):
        ch = flag_s[0]
        n_in = jnp.where(ch > 0.5, jnp.int32(INNER_SLOTS), jnp.int32(0))
        lax.fori_loop(0, n_in, pass_slot, 0)
        return 0

    lax.fori_loop(0, OUTER_CAP, outer_round, 0)

    @pl.when(wid < B_IMG)
    def _():
        def red_blk(v, carry):
            best16, any16 = carry
            o = v * L
            kv = keep_a[pl.ds(o, L)]
            vv = VAL[pl.ds(o, L)]
            best16 = jnp.maximum(best16, jnp.where(kv > 0.5, vv, NEG_BIG))
            any16 = jnp.maximum(any16, kv)
            return (best16, any16)

        best16, any16 = lax.fori_loop(
            0, NB, red_blk,
            (jnp.full((L,), NEG_BIG, jnp.float32),
             jnp.zeros((L,), jnp.float32)))
        score = jnp.where(lane_max(any16) > 0.5, lane_max(best16), 0.0)
        out_v[...] = jnp.full((L,), 1.0, jnp.float32) * score
        pltpu.sync_copy(out_v, out_h.at[pl.ds(wid * L, L)])


@jax.jit
def _od2_score(ff, perm, tbp):
    mesh = plsc.VectorSubcoreMesh(core_axis_name="c", subcore_axis_name="s")
    fn = functools.partial(
        pl.kernel,
        mesh=mesh,
        out_type=jax.ShapeDtypeStruct((B_IMG * L,), jnp.float32),
        scratch_types=[
            pltpu.VMEM((N_PAD,), jnp.float32),
            pltpu.VMEM((N_PAD,), jnp.float32),
            pltpu.VMEM((N_PAD,), jnp.float32),
            pltpu.VMEM((N_PAD,), jnp.float32),
            pltpu.VMEM((N_PAD,), jnp.float32),
            pltpu.VMEM((N_PAD,), jnp.float32),
            pltpu.VMEM((N_PAD,), jnp.int32),
            pltpu.VMEM((N_PAD,), jnp.float32),
            pltpu.VMEM((N_PAD,), jnp.float32),
            pltpu.VMEM((N_PAD,), jnp.float32),
            pltpu.VMEM((N_PAD,), jnp.float32),
            pltpu.VMEM((N_PAD,), jnp.float32),
            pltpu.VMEM((N_PAD,), jnp.float32),
            pltpu.VMEM((N_PAD,), jnp.int32),
            pltpu.VMEM((N_PAD,), jnp.float32),
            pltpu.VMEM((N_PAD,), jnp.float32),
            pltpu.VMEM((N_PAD,), jnp.float32),
            pltpu.VMEM((N_PAD,), jnp.float32),
            pltpu.VMEM((N_PAD,), jnp.float32),
            pltpu.VMEM((L,), jnp.float32),
            pltpu.VMEM((L,), jnp.float32),
            pltpu.VMEM((L,), jnp.float32),
            pltpu.VMEM((NSLICE * L,), jnp.float32),
            pltpu.VMEM_SHARED((2, N_PAD), jnp.float32),
            pltpu.VMEM_SHARED((2, N_PAD), jnp.float32),
            pltpu.VMEM_SHARED((NSLICE * L,), jnp.float32),
            pltpu.SMEM((NB,), jnp.int32),
            pltpu.SMEM((NB,), jnp.int32),
            pltpu.SMEM((NB,), jnp.float32),
            pltpu.SMEM((NB,), jnp.float32),
            pltpu.SMEM((8,), jnp.float32),
        ],
    )(_nms_body)
    return fn(ff, perm, tbp)


def kernel(prediction, target_bbox):
    assert prediction.shape == (B_IMG, N_IN, 6)
    pad = N_PAD - N_IN
    x1 = jnp.pad(prediction[..., 0], ((0, 0), (0, pad)))
    y1 = jnp.pad(prediction[..., 1], ((0, 0), (0, pad)))
    x2 = jnp.pad(prediction[..., 2], ((0, 0), (0, pad)))
    y2 = jnp.pad(prediction[..., 3], ((0, 0), (0, pad)))
    cf = jnp.pad(prediction[..., 4], ((0, 0), (0, pad)), constant_values=-1.0)
    cl = jnp.pad(prediction[..., 5], ((0, 0), (0, pad)), constant_values=1.0e6)
    key = x1 + cl * MAX_WH
    perm = jnp.argsort(key, axis=1).astype(jnp.int32)
    fields = jnp.stack([x1, y1, x2, y2, cf, cl], axis=1)
    fields = jnp.take_along_axis(fields, perm[:, None, :], axis=2)
    tbp = jnp.pad(target_bbox[0].astype(jnp.float32), (0, L - 4))
    out = _od2_score(fields.reshape(-1), perm.reshape(-1), tbp)
    return out.reshape(B_IMG, L)[:, 0]
